# baseline (speedup 1.0000x reference)
import functools

import jax
import jax.numpy as jnp
from jax import lax
from jax.experimental import pallas as pl
from jax.experimental.pallas import tpu as pltpu
from jax.experimental.pallas import tpu_sc as plsc

N = 10000
F = 128
C = 64
WD = 128
G = 8
LIN = 128
OUT = 10
E = 320000
ETOT = E + N

NC = 2
NS = 16
NW = NC * NS
CHUNK = 128
NCHUNK = 82
EPW = NCHUNK * CHUNK
ETOT_PAD = EPW * NW
N_PAD = 10240
ROWS_PT = N_PAD // NS


def _tc_proj(x, w, attd):
    def body(x_ref, w_ref, attd_ref, h_ref, ad_ref, ms_ref):
        h = jnp.dot(x_ref[...], w_ref[...], preferred_element_type=jnp.float32)
        a_d = jnp.sum(h * attd_ref[...], axis=1)
        h_ref[...] = h
        ad_ref[...] = a_d
        ms_ref[...] = jnp.broadcast_to(jnp.max(h[:, C + 1:C + 2]), (128,))

    return pl.pallas_call(
        body,
        out_shape=[
            jax.ShapeDtypeStruct((N, WD), jnp.float32),
            jax.ShapeDtypeStruct((N,), jnp.float32),
            jax.ShapeDtypeStruct((128,), jnp.float32),
        ],
    )(x, w, attd)


def _tc_combine_proj(parts, bias, w, attd):
    def body(p_ref, b_ref, w_ref, attd_ref,
             h_ref, ad_ref, ms_ref):
        comb = (p_ref[0] + p_ref[1])[:N]
        den = comb[:, C:C + 1] + 1e-16
        o = comb[:, :C] / den + b_ref[...]
        x2 = jnp.maximum(o, 0.0)
        h = jnp.dot(x2, w_ref[...], preferred_element_type=jnp.float32)
        a_d = jnp.sum(h * attd_ref[...], axis=1)
        h_ref[...] = h
        ad_ref[...] = a_d
        ms_ref[...] = jnp.broadcast_to(jnp.max(h[:, C + 1:C + 2]), (128,))

    return pl.pallas_call(
        body,
        out_shape=[
            jax.ShapeDtypeStruct((N, WD), jnp.float32),
            jax.ShapeDtypeStruct((N,), jnp.float32),
            jax.ShapeDtypeStruct((128,), jnp.float32),
        ],
    )(parts, bias, w, attd)


def _tc_head(parts, bias, batch, linW, linb, outW, outb):
    def body(p_ref, b_ref, batch_ref, lw_ref, lb_ref, ow_ref, ob_ref,
             out_ref):
        comb = (p_ref[0] + p_ref[1])[:N]
        den = comb[:, C:C + 1] + 1e-16
        o = comb[:, :C] / den + b_ref[...]
        o = jnp.maximum(o, 0.0)
        b = batch_ref[...]
        rows = []
        for g in range(G):
            m = (b == g)
            rows.append(jnp.max(jnp.where(m, o, -jnp.inf), axis=0,
                                keepdims=True))
        gm = jnp.concatenate(rows, axis=0)
        g1 = jnp.dot(gm, lw_ref[...], preferred_element_type=jnp.float32)
        g1 = g1 + lb_ref[...]
        out = jnp.dot(g1, ow_ref[...], preferred_element_type=jnp.float32)
        out_ref[...] = out + ob_ref[...]

    return pl.pallas_call(
        body,
        out_shape=jax.ShapeDtypeStruct((G, OUT), jnp.float32),
    )(parts, bias, batch, linW, linb, outW, outb)


def _sc_edge_pass(h, a_dst, msvec, ei4, z_rows):
    mesh = plsc.VectorSubcoreMesh(core_axis_name="c", subcore_axis_name="s")

    @functools.partial(
        pl.kernel,
        out_type=jax.ShapeDtypeStruct((NC, N_PAD, WD), jnp.float32),
        mesh=mesh,
        compiler_params=pltpu.CompilerParams(needs_layout_passes=False),
        scratch_types=[
            pltpu.VMEM((2, 2, 128), jnp.int32),
            pltpu.VMEM((N,), jnp.float32),
            pltpu.VMEM((16,), jnp.float32),
            pltpu.VMEM((2, CHUNK, WD), jnp.float32),
            pltpu.MemorySpace.VMEM_SHARED((N_PAD, WD), jnp.float32),
            pltpu.SemaphoreType.DMA,
            pltpu.SemaphoreType.DMA,
        ],
    )
    def k(h_hbm, ad_hbm, ms_hbm, ei_hbm, zr_hbm,
          outp_hbm,
          ei_v, ad_v, ms_v, rows_v, acc_sh, semr0, semr1):
        semr = (semr0, semr1)
        c = lax.axis_index("c")
        s = lax.axis_index("s")
        wid = s * NC + c
        pltpu.sync_copy(ad_hbm, ad_v)
        pltpu.sync_copy(ms_hbm.at[pl.ds(0, 16)], ms_v)
        pltpu.sync_copy(zr_hbm.at[pl.ds(s * ROWS_PT, ROWS_PT)],
                        acc_sh.at[pl.ds(s * ROWS_PT, ROWS_PT)])
        plsc.subcore_barrier()

        ebase = wid * EPW
        iota16 = lax.iota(jnp.int32, 16)
        col64 = jnp.full((16,), C, jnp.int32)
        col65 = jnp.full((16,), C + 1, jnp.int32)

        def start_gather(buf):
            pltpu.async_copy(h_hbm.at[ei_v.at[buf, 0]], rows_v.at[buf],
                             semr[buf])

        def wait_gather(buf):
            pltpu.make_async_copy(h_hbm.at[ei_v.at[buf, 0]], rows_v.at[buf],
                                  semr[buf]).wait()

        def compute_chunk(j, buf):
            ms16 = ms_v[...]
            rows_b = rows_v.at[buf]

            def group_body(o, carry):
                rowg = o * 16 + iota16
                a_s = plsc.load_gather(rows_b, [rowg, col65])
                dstg = ei_v[buf, 1, pl.ds(o * 16, 16)]
                a_d = plsc.load_gather(ad_v, [dstg])
                kk = ms16 + a_d
                kk = jnp.where(kk >= 0, kk, 0.2 * kk)
                al = a_s + a_d
                al = jnp.where(al >= 0, al, 0.2 * al)
                al = jnp.exp(al - kk)
                pos = ebase + j * CHUNK + o * 16 + iota16
                al = jnp.where(pos < ETOT, al, 0.0)
                plsc.store_scatter(rows_b, [rowg, col64], al)
                for e in range(16):
                    a = al[e]
                    for cg in range(C // 16):
                        sl = pl.ds(cg * 16, 16)
                        rows_v[buf, o * 16 + e, sl] = \
                            rows_v[buf, o * 16 + e, sl] * a
                return carry

            lax.fori_loop(0, 8, group_body, 0)

        def step(j, buf):
            pltpu.sync_copy(ei_hbm.at[wid, j + 1], ei_v.at[1 - buf])
            wait_gather(buf)
            start_gather(1 - buf)
            compute_chunk(j, buf)
            pltpu.sync_copy(rows_v.at[buf], acc_sh.at[ei_v.at[buf, 1]],
                            add=True)

        pltpu.sync_copy(ei_hbm.at[wid, 0], ei_v.at[0])
        start_gather(0)

        def pair_body(p, carry):
            step(2 * p, 0)
            step(2 * p + 1, 1)
            return carry

        lax.fori_loop(0, NCHUNK // 2, pair_body, 0)
        wait_gather(0)
        plsc.subcore_barrier()
        pltpu.sync_copy(acc_sh.at[pl.ds(s * ROWS_PT, ROWS_PT)],
                        outp_hbm.at[c, pl.ds(s * ROWS_PT, ROWS_PT)])

    return k(h, a_dst, msvec, ei4, z_rows)


def kernel(x, edge_index, batch, W1, att_src1, att_dst1, b1,
           W2, att_src2, att_dst2, b2, linW, linb, outW, outb):
    loop = jnp.arange(N, dtype=edge_index.dtype)
    src = jnp.concatenate([edge_index[0], loop])
    dst = jnp.concatenate([edge_index[1], loop])
    pad = ETOT_PAD - ETOT
    zpad = jnp.zeros((pad,), dtype=src.dtype)
    srcr = jnp.concatenate([src, zpad]).reshape(NW, NCHUNK, 1, 128)
    dstr = jnp.concatenate([dst, zpad]).reshape(NW, NCHUNK, 1, 128)
    ei4 = jnp.concatenate([srcr, dstr], axis=2)
    ei4 = jnp.pad(ei4, ((0, 0), (0, 1), (0, 0), (0, 0)))
    z_rows = jnp.zeros((N_PAD, WD), jnp.float32)

    def aug_w(wmat, att_s):
        acol = wmat @ att_s.reshape(C, 1)
        zcol = jnp.zeros_like(acol)
        tail = jnp.zeros((wmat.shape[0], WD - C - 2), wmat.dtype)
        return jnp.concatenate([wmat, zcol, acol, tail], axis=1)

    W1p = aug_w(W1, att_src1)
    W2p = aug_w(W2, att_src2)
    attd1 = jnp.pad(att_dst1.reshape(1, C), ((0, 0), (0, WD - C)))
    attd2 = jnp.pad(att_dst2.reshape(1, C), ((0, 0), (0, WD - C)))
    batch2 = batch.reshape(N, 1)
    b1r = b1.reshape(1, C)
    b2r = b2.reshape(1, C)
    linbr = linb.reshape(1, LIN)
    outbr = outb.reshape(1, OUT)

    h1, ad1, ms1 = _tc_proj(x, W1p, attd1)
    p1 = _sc_edge_pass(h1, ad1, ms1, ei4, z_rows)
    h2, ad2, ms2 = _tc_combine_proj(p1, b1r, W2p, attd2)
    p2 = _sc_edge_pass(h2, ad2, ms2, ei4, z_rows)
    return _tc_head(p2, b2r, batch2, linW, linbr, outW, outbr)


# speedup vs baseline: 1.0809x; 1.0809x over previous
import functools

import jax
import jax.numpy as jnp
from jax import lax
from jax.experimental import pallas as pl
from jax.experimental.pallas import tpu as pltpu
from jax.experimental.pallas import tpu_sc as plsc

N = 10000
F = 128
C = 64
WD = 128
G = 8
LIN = 128
OUT = 10
E = 320000
ETOT = E + N

NC = 2
NS = 16
NW = NC * NS
CHUNK = 128
NCHUNK = 82
EPW = NCHUNK * CHUNK
ETOT_PAD = EPW * NW
N_PAD = 10240
ROWS_PT = N_PAD // NS


def _tc_proj(x, w, attd):
    def body(x_ref, w_ref, attd_ref, h_ref, ad_ref, ms_ref):
        h = jnp.dot(x_ref[...], w_ref[...], preferred_element_type=jnp.float32)
        a_d = jnp.sum(h * attd_ref[...], axis=1)
        h_ref[...] = h
        ad_ref[...] = a_d
        ms_ref[...] = jnp.broadcast_to(jnp.max(h[:, C + 1:C + 2]), (128,))

    return pl.pallas_call(
        body,
        out_shape=[
            jax.ShapeDtypeStruct((N, WD), jnp.float32),
            jax.ShapeDtypeStruct((N,), jnp.float32),
            jax.ShapeDtypeStruct((128,), jnp.float32),
        ],
    )(x, w, attd)


def _tc_combine_proj(parts, bias, w, attd):
    def body(p_ref, b_ref, w_ref, attd_ref,
             h_ref, ad_ref, ms_ref):
        comb = (p_ref[0] + p_ref[1])[:N]
        den = comb[:, C:C + 1] + 1e-16
        o = comb[:, :C] / den + b_ref[...]
        x2 = jnp.maximum(o, 0.0)
        h = jnp.dot(x2, w_ref[...], preferred_element_type=jnp.float32)
        a_d = jnp.sum(h * attd_ref[...], axis=1)
        h_ref[...] = h
        ad_ref[...] = a_d
        ms_ref[...] = jnp.broadcast_to(jnp.max(h[:, C + 1:C + 2]), (128,))

    return pl.pallas_call(
        body,
        out_shape=[
            jax.ShapeDtypeStruct((N, WD), jnp.float32),
            jax.ShapeDtypeStruct((N,), jnp.float32),
            jax.ShapeDtypeStruct((128,), jnp.float32),
        ],
    )(parts, bias, w, attd)


def _tc_head(parts, bias, batch, linW, linb, outW, outb):
    def body(p_ref, b_ref, batch_ref, lw_ref, lb_ref, ow_ref, ob_ref,
             out_ref):
        comb = (p_ref[0] + p_ref[1])[:N]
        den = comb[:, C:C + 1] + 1e-16
        o = comb[:, :C] / den + b_ref[...]
        o = jnp.maximum(o, 0.0)
        b = batch_ref[...]
        rows = []
        for g in range(G):
            m = (b == g)
            rows.append(jnp.max(jnp.where(m, o, -jnp.inf), axis=0,
                                keepdims=True))
        gm = jnp.concatenate(rows, axis=0)
        g1 = jnp.dot(gm, lw_ref[...], preferred_element_type=jnp.float32)
        g1 = g1 + lb_ref[...]
        out = jnp.dot(g1, ow_ref[...], preferred_element_type=jnp.float32)
        out_ref[...] = out + ob_ref[...]

    return pl.pallas_call(
        body,
        out_shape=jax.ShapeDtypeStruct((G, OUT), jnp.float32),
    )(parts, bias, batch, linW, linb, outW, outb)


def _sc_edge_pass(h, a_dst, msvec, ei4, z_rows):
    mesh = plsc.VectorSubcoreMesh(core_axis_name="c", subcore_axis_name="s")

    @functools.partial(
        pl.kernel,
        out_type=jax.ShapeDtypeStruct((NC, N_PAD, WD), jnp.float32),
        mesh=mesh,
        compiler_params=pltpu.CompilerParams(needs_layout_passes=False),
        scratch_types=[
            pltpu.VMEM((2, 128), jnp.int32),
            pltpu.VMEM((N,), jnp.float32),
            pltpu.VMEM((16,), jnp.float32),
            pltpu.VMEM((CHUNK, WD), jnp.float32),
            pltpu.MemorySpace.VMEM_SHARED((N_PAD, WD), jnp.float32),
            pltpu.SemaphoreType.DMA,
        ],
    )
    def k(h_hbm, ad_hbm, ms_hbm, ei_hbm, zr_hbm,
          outp_hbm,
          ei_v, ad_v, ms_v, rows_v, acc_sh, sem):
        c = lax.axis_index("c")
        s = lax.axis_index("s")
        wid = s * NC + c
        pltpu.sync_copy(ad_hbm, ad_v)
        pltpu.sync_copy(ms_hbm.at[pl.ds(0, 16)], ms_v)
        pltpu.sync_copy(zr_hbm.at[pl.ds(s * ROWS_PT, ROWS_PT)],
                        acc_sh.at[pl.ds(s * ROWS_PT, ROWS_PT)])
        plsc.subcore_barrier()

        ebase = wid * EPW
        iota16 = lax.iota(jnp.int32, 16)
        col64 = jnp.full((16,), C, jnp.int32)
        col65 = jnp.full((16,), C + 1, jnp.int32)

        def chunk_body(j, carry):
            pltpu.sync_copy(ei_hbm.at[wid, j], ei_v)
            pltpu.async_copy(h_hbm.at[ei_v.at[0]], rows_v, sem).wait()
            ms16 = ms_v[...]

            def group_body(o, carry2):
                rowg = o * 16 + iota16
                a_s = plsc.load_gather(rows_v, [rowg, col65])
                dstg = ei_v[1, pl.ds(o * 16, 16)]
                a_d = plsc.load_gather(ad_v, [dstg])
                kk = ms16 + a_d
                kk = jnp.where(kk >= 0, kk, 0.2 * kk)
                al = a_s + a_d
                al = jnp.where(al >= 0, al, 0.2 * al)
                al = jnp.exp(al - kk)
                pos = ebase + j * CHUNK + o * 16 + iota16
                al = jnp.where(pos < ETOT, al, 0.0)
                plsc.store_scatter(rows_v, [rowg, col64], al)
                for e in range(16):
                    a = al[e]
                    for cg in range(C // 16):
                        sl = pl.ds(cg * 16, 16)
                        rows_v[o * 16 + e, sl] = rows_v[o * 16 + e, sl] * a
                return carry2

            lax.fori_loop(0, 8, group_body, 0)
            pltpu.sync_copy(rows_v, acc_sh.at[ei_v.at[1]], add=True)
            return carry

        lax.fori_loop(0, NCHUNK, chunk_body, 0)
        plsc.subcore_barrier()
        pltpu.sync_copy(acc_sh.at[pl.ds(s * ROWS_PT, ROWS_PT)],
                        outp_hbm.at[c, pl.ds(s * ROWS_PT, ROWS_PT)])

    return k(h, a_dst, msvec, ei4, z_rows)


def kernel(x, edge_index, batch, W1, att_src1, att_dst1, b1,
           W2, att_src2, att_dst2, b2, linW, linb, outW, outb):
    loop = jnp.arange(N, dtype=edge_index.dtype)
    src = jnp.concatenate([edge_index[0], loop])
    dst = jnp.concatenate([edge_index[1], loop])
    pad = ETOT_PAD - ETOT
    zpad = jnp.zeros((pad,), dtype=src.dtype)
    srcr = jnp.concatenate([src, zpad]).reshape(NW, NCHUNK, 1, 128)
    dstr = jnp.concatenate([dst, zpad]).reshape(NW, NCHUNK, 1, 128)
    ei4 = jnp.concatenate([srcr, dstr], axis=2)
    ei4 = jnp.pad(ei4, ((0, 0), (0, 1), (0, 0), (0, 0)))
    z_rows = jnp.zeros((N_PAD, WD), jnp.float32)

    def aug_w(wmat, att_s):
        acol = wmat @ att_s.reshape(C, 1)
        zcol = jnp.zeros_like(acol)
        tail = jnp.zeros((wmat.shape[0], WD - C - 2), wmat.dtype)
        return jnp.concatenate([wmat, zcol, acol, tail], axis=1)

    W1p = aug_w(W1, att_src1)
    W2p = aug_w(W2, att_src2)
    attd1 = jnp.pad(att_dst1.reshape(1, C), ((0, 0), (0, WD - C)))
    attd2 = jnp.pad(att_dst2.reshape(1, C), ((0, 0), (0, WD - C)))
    batch2 = batch.reshape(N, 1)
    b1r = b1.reshape(1, C)
    b2r = b2.reshape(1, C)
    linbr = linb.reshape(1, LIN)
    outbr = outb.reshape(1, OUT)

    h1, ad1, ms1 = _tc_proj(x, W1p, attd1)
    p1 = _sc_edge_pass(h1, ad1, ms1, ei4, z_rows)
    h2, ad2, ms2 = _tc_combine_proj(p1, b1r, W2p, attd2)
    p2 = _sc_edge_pass(h2, ad2, ms2, ei4, z_rows)
    return _tc_head(p2, b2r, batch2, linW, linbr, outW, outbr)


# speedup vs baseline: 1.5342x; 1.4193x over previous
import functools

import jax
import jax.numpy as jnp
from jax import lax
from jax.experimental import pallas as pl
from jax.experimental.pallas import tpu as pltpu
from jax.experimental.pallas import tpu_sc as plsc

N = 10000
F = 128
C = 64
WD = 128
G = 8
LIN = 128
OUT = 10
E = 320000
ETOT = E + N

NC = 2
NS = 16
NW = NC * NS
CHUNK = 128
NCHUNK = -(-ETOT // (NW * CHUNK))
EPW = NCHUNK * CHUNK
ETOT_PAD = EPW * NW
N_PAD = 10240
ROWS_PT = N_PAD // NS


def _tc_proj(x, w, attd):
    def body(x_ref, w_ref, attd_ref, h_ref, ad_ref, ms_ref):
        h = jnp.dot(x_ref[...], w_ref[...], preferred_element_type=jnp.float32)
        a_d = jnp.sum(h * attd_ref[...], axis=1)
        h_ref[...] = h
        ad_ref[...] = a_d
        ms_ref[...] = jnp.broadcast_to(jnp.max(h[:, C + 1:C + 2]), (128,))

    return pl.pallas_call(
        body,
        out_shape=[
            jax.ShapeDtypeStruct((N, WD), jnp.float32),
            jax.ShapeDtypeStruct((N,), jnp.float32),
            jax.ShapeDtypeStruct((128,), jnp.float32),
        ],
    )(x, w, attd)


def _tc_combine_proj(parts, bias, w, attd):
    def body(p_ref, b_ref, w_ref, attd_ref,
             h_ref, ad_ref, ms_ref):
        comb = (p_ref[0] + p_ref[1])[:N]
        den = comb[:, C:C + 1] + 1e-16
        o = comb[:, :C] / den + b_ref[...]
        x2 = jnp.maximum(o, 0.0)
        h = jnp.dot(x2, w_ref[...], preferred_element_type=jnp.float32)
        a_d = jnp.sum(h * attd_ref[...], axis=1)
        h_ref[...] = h
        ad_ref[...] = a_d
        ms_ref[...] = jnp.broadcast_to(jnp.max(h[:, C + 1:C + 2]), (128,))

    return pl.pallas_call(
        body,
        out_shape=[
            jax.ShapeDtypeStruct((N, WD), jnp.float32),
            jax.ShapeDtypeStruct((N,), jnp.float32),
            jax.ShapeDtypeStruct((128,), jnp.float32),
        ],
    )(parts, bias, w, attd)


def _tc_head(parts, bias, batch, linW, linb, outW, outb):
    def body(p_ref, b_ref, batch_ref, lw_ref, lb_ref, ow_ref, ob_ref,
             out_ref):
        comb = (p_ref[0] + p_ref[1])[:N]
        den = comb[:, C:C + 1] + 1e-16
        o = comb[:, :C] / den + b_ref[...]
        o = jnp.maximum(o, 0.0)
        b = batch_ref[...]
        rows = []
        for g in range(G):
            m = (b == g)
            rows.append(jnp.max(jnp.where(m, o, -jnp.inf), axis=0,
                                keepdims=True))
        gm = jnp.concatenate(rows, axis=0)
        g1 = jnp.dot(gm, lw_ref[...], preferred_element_type=jnp.float32)
        g1 = g1 + lb_ref[...]
        out = jnp.dot(g1, ow_ref[...], preferred_element_type=jnp.float32)
        out_ref[...] = out + ob_ref[...]

    return pl.pallas_call(
        body,
        out_shape=jax.ShapeDtypeStruct((G, OUT), jnp.float32),
    )(parts, bias, batch, linW, linb, outW, outb)


def _sc_edge_pass(h, a_dst, msvec, src3, dst3, z_rows):
    mesh = plsc.VectorSubcoreMesh(core_axis_name="c", subcore_axis_name="s")

    @functools.partial(
        pl.kernel,
        out_type=jax.ShapeDtypeStruct((NC, N_PAD, WD), jnp.float32),
        mesh=mesh,
        compiler_params=pltpu.CompilerParams(needs_layout_passes=False),
        scratch_types=[
            pltpu.VMEM((CHUNK,), jnp.int32),
            pltpu.VMEM((CHUNK,), jnp.int32),
            pltpu.VMEM((N,), jnp.float32),
            pltpu.VMEM((16,), jnp.float32),
            pltpu.VMEM((CHUNK,), jnp.float32),
            pltpu.VMEM((CHUNK, WD), jnp.float32),
            pltpu.MemorySpace.VMEM_SHARED((N_PAD, WD), jnp.float32),
            pltpu.SemaphoreType.DMA,
        ],
    )
    def k(h_hbm, ad_hbm, ms_hbm, src_hbm, dst_hbm, zr_hbm,
          outp_hbm,
          src_v, dst_v, ad_v, ms_v, alpha_v, rows_v, acc_sh, sem):
        c = lax.axis_index("c")
        s = lax.axis_index("s")
        wid = s * NC + c
        pltpu.sync_copy(ad_hbm, ad_v)
        pltpu.sync_copy(ms_hbm.at[pl.ds(0, 16)], ms_v)
        pltpu.sync_copy(zr_hbm.at[pl.ds(s * ROWS_PT, ROWS_PT)],
                        acc_sh.at[pl.ds(s * ROWS_PT, ROWS_PT)])
        plsc.subcore_barrier()

        ebase = wid * EPW
        iota16 = lax.iota(jnp.int32, 16)

        def chunk_body(j, carry):
            pltpu.sync_copy(src_hbm.at[wid, j], src_v)
            pltpu.sync_copy(dst_hbm.at[wid, j], dst_v)
            pltpu.async_copy(h_hbm.at[src_v], rows_v, sem).wait()
            ms16 = ms_v[...]
            for o in range(CHUNK // 16):
                rowg = o * 16 + iota16
                col65 = jnp.full((16,), C + 1, jnp.int32)
                a_s = plsc.load_gather(rows_v, [rowg, col65])
                dstg = dst_v[pl.ds(o * 16, 16)]
                a_d = plsc.load_gather(ad_v, [dstg])
                kk = ms16 + a_d
                kk = jnp.where(kk >= 0, kk, 0.2 * kk)
                al = a_s + a_d
                al = jnp.where(al >= 0, al, 0.2 * al)
                al = jnp.exp(al - kk)
                pos = ebase + j * CHUNK + o * 16 + iota16
                al = jnp.where(pos < ETOT, al, 0.0)
                alpha_v[pl.ds(o * 16, 16)] = al
            for o in range(CHUNK // 16):
                al16 = alpha_v[pl.ds(o * 16, 16)]
                for e in range(16):
                    a = al16[e]
                    row = o * 16 + e
                    for cg in range(5):
                        sl = pl.ds(cg * 16, 16)
                        rows_v[row, sl] = rows_v[row, sl] * a
            pltpu.sync_copy(rows_v, acc_sh.at[dst_v], add=True)
            return carry

        lax.fori_loop(0, NCHUNK, chunk_body, 0)
        plsc.subcore_barrier()
        pltpu.sync_copy(acc_sh.at[pl.ds(s * ROWS_PT, ROWS_PT)],
                        outp_hbm.at[c, pl.ds(s * ROWS_PT, ROWS_PT)])

    return k(h, a_dst, msvec, src3, dst3, z_rows)


def kernel(x, edge_index, batch, W1, att_src1, att_dst1, b1,
           W2, att_src2, att_dst2, b2, linW, linb, outW, outb):
    loop = jnp.arange(N, dtype=edge_index.dtype)
    src = jnp.concatenate([edge_index[0], loop])
    dst = jnp.concatenate([edge_index[1], loop])
    pad = ETOT_PAD - ETOT
    zpad = jnp.zeros((pad,), dtype=src.dtype)
    src3 = jnp.concatenate([src, zpad]).reshape(NW, NCHUNK, CHUNK)
    dst3 = jnp.concatenate([dst, zpad]).reshape(NW, NCHUNK, CHUNK)
    z_rows = jnp.zeros((N_PAD, WD), jnp.float32)

    def aug_w(wmat, att_s):
        acol = wmat @ att_s.reshape(C, 1)
        zcol = jnp.zeros_like(acol)
        tail = jnp.zeros((wmat.shape[0], WD - C - 2), wmat.dtype)
        return jnp.concatenate([wmat, zcol, acol, tail], axis=1)

    W1p = aug_w(W1, att_src1)
    W2p = aug_w(W2, att_src2)
    attd1 = jnp.pad(att_dst1.reshape(1, C), ((0, 0), (0, WD - C)))
    attd2 = jnp.pad(att_dst2.reshape(1, C), ((0, 0), (0, WD - C)))
    batch2 = batch.reshape(N, 1)
    b1r = b1.reshape(1, C)
    b2r = b2.reshape(1, C)
    linbr = linb.reshape(1, LIN)
    outbr = outb.reshape(1, OUT)

    h1, ad1, ms1 = _tc_proj(x, W1p, attd1)
    p1 = _sc_edge_pass(h1, ad1, ms1, src3, dst3, z_rows)
    h2, ad2, ms2 = _tc_combine_proj(p1, b1r, W2p, attd2)
    p2 = _sc_edge_pass(h2, ad2, ms2, src3, dst3, z_rows)
    return _tc_head(p2, b2r, batch2, linW, linbr, outW, outbr)
